# baseline (speedup 1.0000x reference)
import functools

import jax
import jax.numpy as jnp
from jax import lax
from jax.experimental import pallas as pl
from jax.experimental.pallas import tpu as pltpu
from jax.experimental.pallas import tpu_sc as plsc

N = 10000
NE = 10000
E = 320000
D = 128

NC = 2
NS = 16
NW = NC * NS
T = E // NW
K = 80
T_PAD = 10000
NB = T_PAD // K
NPH = 5
PB = NB // NPH
S_PAD = 10240
ROWS_PER_TILE = S_PAD // NS


def _agg_body(table, gidx, sidx, zeros, out, gidx_v, sidx_v, rows_v, acc_sh,
              sig, sis, sg0, sg1):
    c = lax.axis_index("c")
    s = lax.axis_index("s")
    wid = c * NS + s
    gsems = (sg0, sg1)
    pltpu.sync_copy(gidx.at[wid, 0], gidx_v.at[0])
    pltpu.sync_copy(sidx.at[wid, 0], sidx_v.at[0])
    pltpu.sync_copy(zeros, rows_v.at[0])
    for r in range(ROWS_PER_TILE // K):
        pltpu.sync_copy(rows_v.at[0],
                        acc_sh.at[pl.ds(s * ROWS_PER_TILE + r * K, K)])
    plsc.subcore_barrier()

    for ph in range(NPH):
        ib = ph & 1
        if ph + 1 < NPH:
            pltpu.async_copy(gidx.at[wid, ph + 1], gidx_v.at[1 - ib], sig)
            pltpu.async_copy(sidx.at[wid, ph + 1], sidx_v.at[1 - ib], sis)

        def gstart(j, b, ib=ib):
            pltpu.async_copy(table.at[gidx_v.at[ib, j]], rows_v.at[b],
                             gsems[b])

        def gwait(b):
            pltpu.make_async_copy(table.at[gidx_v.at[0, 0]], rows_v.at[b],
                                  gsems[b]).wait()

        def scat(j, b, ib=ib):
            pltpu.sync_copy(rows_v.at[b], acc_sh.at[sidx_v.at[ib, j]],
                            add=True)

        gstart(0, 0)

        def pair(i, carry):
            gstart(2 * i + 1, 1)
            gwait(0)
            scat(2 * i, 0)
            gstart(2 * i + 2, 0)
            gwait(1)
            scat(2 * i + 1, 1)
            return carry

        lax.fori_loop(0, (PB - 1) // 2, pair, 0)
        gwait(0)
        scat(PB - 1, 0)

        if ph + 1 < NPH:
            pltpu.make_async_copy(gidx.at[0, 0], gidx_v.at[1 - ib], sig).wait()
            pltpu.make_async_copy(sidx.at[0, 0], sidx_v.at[1 - ib], sis).wait()

    plsc.subcore_barrier()
    for r in range(ROWS_PER_TILE // K):
        sl = pl.ds(s * ROWS_PER_TILE + r * K, K)
        pltpu.sync_copy(acc_sh.at[sl], rows_v.at[0])
        pltpu.sync_copy(rows_v.at[0], out.at[c, sl])


def _make_agg():
    mesh = plsc.VectorSubcoreMesh(core_axis_name="c", subcore_axis_name="s")
    return pl.kernel(
        _agg_body,
        out_type=jax.ShapeDtypeStruct((NC, S_PAD, D), jnp.float32),
        mesh=mesh,
        scratch_types=[
            pltpu.VMEM((2, PB, K), jnp.int32),
            pltpu.VMEM((2, PB, K), jnp.int32),
            pltpu.VMEM((2, K, D), jnp.float32),
            pltpu.VMEM_SHARED((S_PAD, D), jnp.float32),
            pltpu.SemaphoreType.DMA,
            pltpu.SemaphoreType.DMA,
            pltpu.SemaphoreType.DMA,
            pltpu.SemaphoreType.DMA,
        ],
    )


def _combine_body(relu, p_ref, cnt_ref, w_ref, b_ref, o_ref):
    ssum = p_ref[0] + p_ref[1]
    cnt = cnt_ref[0] + cnt_ref[1]
    mean = ssum / jnp.maximum(cnt, 1.0)
    y = jnp.dot(mean, w_ref[...], preferred_element_type=jnp.float32)
    y = y[:NE] + b_ref[...][None, :]
    if relu:
        y = jnp.maximum(y, 0.0)
    o_ref[...] = y


def _combine(partials, cnts, w, b, relu):
    body = functools.partial(_combine_body, relu)
    return pl.pallas_call(
        body,
        out_shape=jax.ShapeDtypeStruct((NE, D), jnp.float32),
    )(partials, cnts, w, b)


def _pad_idx(g, s_):
    return (g.reshape(NW, NPH, PB, K), s_.reshape(NW, NPH, PB, K))


def kernel(x, ei, W1_e, b1_e, W1_n, b1_n, W2_e, b2_e, W2_n, b2_n):
    gi_ne, si_ne = _pad_idx(ei[0], ei[1])
    gi_en, si_en = _pad_idx(ei[1], ei[0])
    zeros_b = jnp.zeros((K, D), jnp.float32)
    ones_t = jnp.ones((N, D), jnp.float32)

    agg = _make_agg()
    gi_z = jnp.zeros((NW, NPH, PB, K), jnp.int32)
    cnt_e = agg(ones_t, gi_z, si_ne, zeros_b)
    cnt_n = agg(ones_t, gi_z, si_en, zeros_b)

    h = x
    for (We, be, Wn, bn) in ((W1_e, b1_e, W1_n, b1_n), (W2_e, b2_e, W2_n, b2_n)):
        ep = agg(h, gi_ne, si_ne, zeros_b)
        ef = _combine(ep, cnt_e, We, be, relu=False)
        np_ = agg(ef, gi_en, si_en, zeros_b)
        h = _combine(np_, cnt_n, Wn, bn, relu=True)
    return h


# speedup vs baseline: 24.4125x; 24.4125x over previous
import functools

import jax
import jax.numpy as jnp
from jax import lax
from jax.experimental import pallas as pl
from jax.experimental.pallas import tpu as pltpu
from jax.experimental.pallas import tpu_sc as plsc

N = 10000
NE = 10000
E = 320000
D = 128

NC = 2
NS = 16
NW = NC * NS
T = E // NW
K = 80
T_PAD = 10000
NB = T_PAD // K
NPH = 5
PB = NB // NPH
S_PAD = 10240
ROWS_PER_TILE = S_PAD // NS


def _agg_body(table, gidx, sidx, zeros, out, gidx_v, sidx_v, rows_v, acc_sh,
              sig, sis, sg0, sg1, ss0, ss1):
    c = lax.axis_index("c")
    s = lax.axis_index("s")
    wid = c * NS + s
    gsems = (sg0, sg1)
    ssems = (ss0, ss1)
    pltpu.sync_copy(gidx.at[wid, 0], gidx_v.at[0])
    pltpu.sync_copy(sidx.at[wid, 0], sidx_v.at[0])
    pltpu.sync_copy(zeros, rows_v.at[0])
    for r in range(ROWS_PER_TILE // K):
        pltpu.sync_copy(rows_v.at[0],
                        acc_sh.at[pl.ds(s * ROWS_PER_TILE + r * K, K)])
    plsc.subcore_barrier()

    for ph in range(NPH):
        ib = ph & 1
        if ph + 1 < NPH:
            pltpu.async_copy(gidx.at[wid, ph + 1], gidx_v.at[1 - ib], sig)
            pltpu.async_copy(sidx.at[wid, ph + 1], sidx_v.at[1 - ib], sis)

        def gstart(j, b, ib=ib):
            pltpu.async_copy(table.at[gidx_v.at[ib, j]], rows_v.at[b],
                             gsems[b])

        def gwait(b):
            pltpu.make_async_copy(table.at[gidx_v.at[0, 0]], rows_v.at[b],
                                  gsems[b]).wait()

        def sstart(j, b, ib=ib):
            pltpu.async_copy(rows_v.at[b], acc_sh.at[sidx_v.at[ib, j]],
                             ssems[b], add=True)

        def swait(b, ib=ib):
            pltpu.make_async_copy(rows_v.at[b], acc_sh.at[sidx_v.at[ib, 0]],
                                  ssems[b]).wait()

        gstart(0, 0)
        gstart(1, 1)

        def pair(i, carry):
            gwait(0)
            sstart(2 * i, 0)
            gwait(1)
            sstart(2 * i + 1, 1)
            swait(0)
            gstart(2 * i + 2, 0)
            swait(1)
            gstart(2 * i + 3, 1)
            return carry

        lax.fori_loop(0, (PB - 3) // 2, pair, 0)
        jl = PB - 3
        gwait(0)
        sstart(jl, 0)
        gwait(1)
        sstart(jl + 1, 1)
        swait(0)
        gstart(jl + 2, 0)
        swait(1)
        gwait(0)
        sstart(jl + 2, 0)
        swait(0)

        if ph + 1 < NPH:
            pltpu.make_async_copy(gidx.at[0, 0], gidx_v.at[1 - ib], sig).wait()
            pltpu.make_async_copy(sidx.at[0, 0], sidx_v.at[1 - ib], sis).wait()

    plsc.subcore_barrier()
    for r in range(ROWS_PER_TILE // K):
        sl = pl.ds(s * ROWS_PER_TILE + r * K, K)
        pltpu.sync_copy(acc_sh.at[sl], rows_v.at[0])
        pltpu.sync_copy(rows_v.at[0], out.at[c, sl])


def _make_agg():
    mesh = plsc.VectorSubcoreMesh(core_axis_name="c", subcore_axis_name="s")
    return pl.kernel(
        _agg_body,
        out_type=jax.ShapeDtypeStruct((NC, S_PAD, D), jnp.float32),
        mesh=mesh,
        scratch_types=[
            pltpu.VMEM((2, PB, K), jnp.int32),
            pltpu.VMEM((2, PB, K), jnp.int32),
            pltpu.VMEM((2, K, D), jnp.float32),
            pltpu.VMEM_SHARED((S_PAD, D), jnp.float32),
            pltpu.SemaphoreType.DMA,
            pltpu.SemaphoreType.DMA,
            pltpu.SemaphoreType.DMA,
            pltpu.SemaphoreType.DMA,
            pltpu.SemaphoreType.DMA,
            pltpu.SemaphoreType.DMA,
        ],
    )


def _combine_body(relu, p_ref, cnt_ref, w_ref, b_ref, o_ref):
    ssum = p_ref[0] + p_ref[1]
    cnt = cnt_ref[0] + cnt_ref[1]
    mean = ssum / jnp.maximum(cnt, 1.0)
    y = jnp.dot(mean, w_ref[...], preferred_element_type=jnp.float32)
    y = y[:NE] + b_ref[...][None, :]
    if relu:
        y = jnp.maximum(y, 0.0)
    o_ref[...] = y


def _combine(partials, cnts, w, b, relu):
    body = functools.partial(_combine_body, relu)
    return pl.pallas_call(
        body,
        out_shape=jax.ShapeDtypeStruct((NE, D), jnp.float32),
    )(partials, cnts, w, b)


def _pad_idx(g, s_):
    return (g.reshape(NW, NPH, PB, K), s_.reshape(NW, NPH, PB, K))


def kernel(x, ei, W1_e, b1_e, W1_n, b1_n, W2_e, b2_e, W2_n, b2_n):
    gi_ne, si_ne = _pad_idx(ei[0], ei[1])
    gi_en, si_en = _pad_idx(ei[1], ei[0])
    zeros_b = jnp.zeros((K, D), jnp.float32)
    ones_t = jnp.ones((N, D), jnp.float32)

    agg = _make_agg()
    cnt_e = agg(ones_t, gi_ne, si_ne, zeros_b)
    cnt_n = agg(ones_t, gi_en, si_en, zeros_b)

    h = x
    for (We, be, Wn, bn) in ((W1_e, b1_e, W1_n, b1_n), (W2_e, b2_e, W2_n, b2_n)):
        ep = agg(h, gi_ne, si_ne, zeros_b)
        ef = _combine(ep, cnt_e, We, be, relu=False)
        np_ = agg(ef, gi_en, si_en, zeros_b)
        h = _combine(np_, cnt_n, Wn, bn, relu=True)
    return h


# speedup vs baseline: 29.2717x; 1.1990x over previous
import functools

import jax
import jax.numpy as jnp
from jax import lax
from jax.experimental import pallas as pl
from jax.experimental.pallas import tpu as pltpu
from jax.experimental.pallas import tpu_sc as plsc

N = 10000
NE = 10000
E = 320000
D = 128

NC = 2
NS = 16
NW = NC * NS
T = E // NW
K = 80
T_PAD = 10000
NB = T_PAD // K
NPH = 5
PB = NB // NPH
S_PAD = 10240
ROWS_PER_TILE = S_PAD // NS


def _agg_body(table, gidx, sidx, zeros, out, gidx_v, sidx_v, rows_v, acc_sh,
              sig, sis, sg0, sg1, ss0, ss1):
    c = lax.axis_index("c")
    s = lax.axis_index("s")
    wid = c * NS + s
    gsems = (sg0, sg1)
    ssems = (ss0, ss1)
    pltpu.sync_copy(gidx.at[wid, 0], gidx_v.at[0])
    pltpu.sync_copy(sidx.at[wid, 0], sidx_v.at[0])
    pltpu.sync_copy(zeros, rows_v.at[0])
    for r in range(ROWS_PER_TILE // K):
        pltpu.sync_copy(rows_v.at[0],
                        acc_sh.at[pl.ds(s * ROWS_PER_TILE + r * K, K)])
    plsc.subcore_barrier()

    for ph in range(NPH):
        ib = ph & 1
        if ph + 1 < NPH:
            pltpu.async_copy(gidx.at[wid, ph + 1], gidx_v.at[1 - ib], sig)
            pltpu.async_copy(sidx.at[wid, ph + 1], sidx_v.at[1 - ib], sis)

        def gstart(j, b, ib=ib):
            pltpu.async_copy(table.at[gidx_v.at[ib, j]], rows_v.at[b],
                             gsems[b])

        def gwait(b):
            pltpu.make_async_copy(table.at[gidx_v.at[0, 0]], rows_v.at[b],
                                  gsems[b]).wait()

        def scat(j, b, ib=ib):
            pltpu.sync_copy(rows_v.at[b], acc_sh.at[sidx_v.at[ib, j]],
                            add=True)

        gstart(0, 0)

        def pair(i, carry):
            gstart(2 * i + 1, 1)
            gwait(0)
            scat(2 * i, 0)
            gstart(2 * i + 2, 0)
            gwait(1)
            scat(2 * i + 1, 1)
            return carry

        lax.fori_loop(0, (PB - 1) // 2, pair, 0)
        gwait(0)
        scat(PB - 1, 0)

        if ph + 1 < NPH:
            pltpu.make_async_copy(gidx.at[0, 0], gidx_v.at[1 - ib], sig).wait()
            pltpu.make_async_copy(sidx.at[0, 0], sidx_v.at[1 - ib], sis).wait()

    plsc.subcore_barrier()
    for r in range(ROWS_PER_TILE // K):
        sl = pl.ds(s * ROWS_PER_TILE + r * K, K)
        pltpu.sync_copy(acc_sh.at[sl], rows_v.at[0])
        pltpu.sync_copy(rows_v.at[0], out.at[c, sl])


def _make_agg():
    mesh = plsc.VectorSubcoreMesh(core_axis_name="c", subcore_axis_name="s")
    return pl.kernel(
        _agg_body,
        out_type=jax.ShapeDtypeStruct((NC, S_PAD, D), jnp.float32),
        mesh=mesh,
        scratch_types=[
            pltpu.VMEM((2, PB, K), jnp.int32),
            pltpu.VMEM((2, PB, K), jnp.int32),
            pltpu.VMEM((2, K, D), jnp.float32),
            pltpu.VMEM_SHARED((S_PAD, D), jnp.float32),
            pltpu.SemaphoreType.DMA,
            pltpu.SemaphoreType.DMA,
            pltpu.SemaphoreType.DMA,
            pltpu.SemaphoreType.DMA,
            pltpu.SemaphoreType.DMA,
            pltpu.SemaphoreType.DMA,
        ],
    )


def _combine_body(relu, p_ref, cnt_ref, w_ref, b_ref, o_ref):
    ssum = p_ref[0] + p_ref[1]
    cnt = cnt_ref[0] + cnt_ref[1]
    mean = ssum / jnp.maximum(cnt, 1.0)
    y = jnp.dot(mean, w_ref[...], preferred_element_type=jnp.float32)
    y = y[:NE] + b_ref[...][None, :]
    if relu:
        y = jnp.maximum(y, 0.0)
    o_ref[...] = y


def _combine(partials, cnts, w, b, relu):
    body = functools.partial(_combine_body, relu)
    return pl.pallas_call(
        body,
        out_shape=jax.ShapeDtypeStruct((NE, D), jnp.float32),
    )(partials, cnts, w, b)


def _pad_idx(g, s_):
    return (g.reshape(NW, NPH, PB, K), s_.reshape(NW, NPH, PB, K))


def kernel(x, ei, W1_e, b1_e, W1_n, b1_n, W2_e, b2_e, W2_n, b2_n):
    gi_ne, si_ne = _pad_idx(ei[0], ei[1])
    gi_en, si_en = _pad_idx(ei[1], ei[0])
    zeros_b = jnp.zeros((K, D), jnp.float32)
    ones_t = jnp.ones((N, D), jnp.float32)

    agg = _make_agg()
    gi_seq = jnp.broadcast_to(
        jnp.arange(T, dtype=jnp.int32).reshape(1, NPH, PB, K),
        (NW, NPH, PB, K))
    cnt_e = agg(ones_t, gi_seq, si_ne, zeros_b)
    cnt_n = agg(ones_t, gi_seq, si_en, zeros_b)

    h = x
    for (We, be, Wn, bn) in ((W1_e, b1_e, W1_n, b1_n), (W2_e, b2_e, W2_n, b2_n)):
        ep = agg(h, gi_ne, si_ne, zeros_b)
        ef = _combine(ep, cnt_e, We, be, relu=False)
        np_ = agg(ef, gi_en, si_en, zeros_b)
        h = _combine(np_, cnt_n, Wn, bn, relu=True)
    return h


# speedup vs baseline: 30.8538x; 1.0540x over previous
import functools

import jax
import jax.numpy as jnp
from jax import lax
from jax.experimental import pallas as pl
from jax.experimental.pallas import tpu as pltpu
from jax.experimental.pallas import tpu_sc as plsc

N = 10000
NE = 10000
E = 320000
D = 128

NC = 2
NS = 16
NW = NC * NS
T = E // NW
K = 80
T_PAD = 10000
NB = T_PAD // K
NPH = 5
PB = NB // NPH
S_PAD = 10240
ROWS_PER_TILE = S_PAD // NS


def _agg_body(table, gidx, sidx, zeros, out, gidx_v, sidx_v, rows_v, acc_sh,
              sig, sis, sg0, sg1, ss0, ss1):
    c = lax.axis_index("c")
    s = lax.axis_index("s")
    wid = c * NS + s
    gsems = (sg0, sg1)
    ssems = (ss0, ss1)
    pltpu.sync_copy(gidx.at[wid, 0], gidx_v.at[0])
    pltpu.sync_copy(sidx.at[wid, 0], sidx_v.at[0])
    pltpu.sync_copy(zeros, rows_v.at[0])
    for r in range(ROWS_PER_TILE // K):
        pltpu.async_copy(rows_v.at[0],
                         acc_sh.at[pl.ds(s * ROWS_PER_TILE + r * K, K)], sig)
    for r in range(ROWS_PER_TILE // K):
        pltpu.make_async_copy(rows_v.at[0],
                              acc_sh.at[pl.ds(s * ROWS_PER_TILE, K)],
                              sig).wait()
    plsc.subcore_barrier()

    for ph in range(NPH):
        ib = ph & 1
        if ph + 1 < NPH:
            pltpu.async_copy(gidx.at[wid, ph + 1], gidx_v.at[1 - ib], sig)
            pltpu.async_copy(sidx.at[wid, ph + 1], sidx_v.at[1 - ib], sis)

        def gstart(j, b, ib=ib):
            pltpu.async_copy(table.at[gidx_v.at[ib, j]], rows_v.at[b],
                             gsems[b])

        def gwait(b):
            pltpu.make_async_copy(table.at[gidx_v.at[0, 0]], rows_v.at[b],
                                  gsems[b]).wait()

        def scat(j, b, ib=ib):
            pltpu.sync_copy(rows_v.at[b], acc_sh.at[sidx_v.at[ib, j]],
                            add=True)

        gstart(0, 0)

        def pair(i, carry):
            gstart(2 * i + 1, 1)
            gwait(0)
            scat(2 * i, 0)
            gstart(2 * i + 2, 0)
            gwait(1)
            scat(2 * i + 1, 1)
            return carry

        lax.fori_loop(0, (PB - 1) // 2, pair, 0)
        gwait(0)
        scat(PB - 1, 0)

        if ph + 1 < NPH:
            pltpu.make_async_copy(gidx.at[0, 0], gidx_v.at[1 - ib], sig).wait()
            pltpu.make_async_copy(sidx.at[0, 0], sidx_v.at[1 - ib], sis).wait()

    plsc.subcore_barrier()
    for r in range(ROWS_PER_TILE // K):
        b = r & 1
        sl = pl.ds(s * ROWS_PER_TILE + r * K, K)
        if r >= 2:
            pltpu.make_async_copy(rows_v.at[b], out.at[c, sl],
                                  gsems[b]).wait()
        pltpu.sync_copy(acc_sh.at[sl], rows_v.at[b])
        pltpu.async_copy(rows_v.at[b], out.at[c, sl], gsems[b])
    for b in range(2):
        pltpu.make_async_copy(rows_v.at[b],
                              out.at[c, pl.ds(s * ROWS_PER_TILE, K)],
                              gsems[b]).wait()


def _make_agg():
    mesh = plsc.VectorSubcoreMesh(core_axis_name="c", subcore_axis_name="s")
    return pl.kernel(
        _agg_body,
        out_type=jax.ShapeDtypeStruct((NC, S_PAD, D), jnp.float32),
        mesh=mesh,
        scratch_types=[
            pltpu.VMEM((2, PB, K), jnp.int32),
            pltpu.VMEM((2, PB, K), jnp.int32),
            pltpu.VMEM((2, K, D), jnp.float32),
            pltpu.VMEM_SHARED((S_PAD, D), jnp.float32),
            pltpu.SemaphoreType.DMA,
            pltpu.SemaphoreType.DMA,
            pltpu.SemaphoreType.DMA,
            pltpu.SemaphoreType.DMA,
            pltpu.SemaphoreType.DMA,
            pltpu.SemaphoreType.DMA,
        ],
    )


def _combine_body(relu, p_ref, cnt_ref, w_ref, b_ref, o_ref):
    ssum = p_ref[0] + p_ref[1]
    cnt = cnt_ref[0] + cnt_ref[1]
    mean = ssum / jnp.maximum(cnt, 1.0)
    y = jnp.dot(mean, w_ref[...], preferred_element_type=jnp.float32)
    y = y[:NE] + b_ref[...][None, :]
    if relu:
        y = jnp.maximum(y, 0.0)
    o_ref[...] = y


def _combine(partials, cnts, w, b, relu):
    body = functools.partial(_combine_body, relu)
    return pl.pallas_call(
        body,
        out_shape=jax.ShapeDtypeStruct((NE, D), jnp.float32),
    )(partials, cnts, w, b)


def _pad_idx(g, s_):
    return (g.reshape(NW, NPH, PB, K), s_.reshape(NW, NPH, PB, K))


def kernel(x, ei, W1_e, b1_e, W1_n, b1_n, W2_e, b2_e, W2_n, b2_n):
    gi_ne, si_ne = _pad_idx(ei[0], ei[1])
    gi_en, si_en = _pad_idx(ei[1], ei[0])
    zeros_b = jnp.zeros((K, D), jnp.float32)
    ones_t = jnp.ones((N, D), jnp.float32)

    agg = _make_agg()
    cnt_e = agg(ones_t, gi_ne, si_ne, zeros_b)
    cnt_n = agg(ones_t, gi_en, si_en, zeros_b)

    h = x
    for (We, be, Wn, bn) in ((W1_e, b1_e, W1_n, b1_n), (W2_e, b2_e, W2_n, b2_n)):
        ep = agg(h, gi_ne, si_ne, zeros_b)
        ef = _combine(ep, cnt_e, We, be, relu=False)
        np_ = agg(ef, gi_en, si_en, zeros_b)
        h = _combine(np_, cnt_n, Wn, bn, relu=True)
    return h


# speedup vs baseline: 30.8968x; 1.0014x over previous
import functools

import jax
import jax.numpy as jnp
from jax import lax
from jax.experimental import pallas as pl
from jax.experimental.pallas import tpu as pltpu
from jax.experimental.pallas import tpu_sc as plsc

N = 10000
NE = 10000
E = 320000
D = 128

NC = 2
NS = 16
NW = NC * NS
T = E // NW
K = 80
T_PAD = 10000
NB = T_PAD // K
NPH = 5
PB = NB // NPH
S_PAD = 10240
ROWS_PER_TILE = S_PAD // NS


def _agg_body(table, gidx, sidx, zeros, out, gidx_v, sidx_v, rows_v, acc_sh,
              sig, sis, sg0, sg1):
    c = lax.axis_index("c")
    s = lax.axis_index("s")
    wid = c * NS + s
    gsems = (sg0, sg1)
    pltpu.sync_copy(gidx.at[wid, 0], gidx_v.at[0])
    pltpu.sync_copy(sidx.at[wid, 0], sidx_v.at[0])
    pltpu.sync_copy(zeros, rows_v.at[0])
    for r in range(ROWS_PER_TILE // K):
        pltpu.async_copy(rows_v.at[0],
                         acc_sh.at[pl.ds(s * ROWS_PER_TILE + r * K, K)], sig)
    for r in range(ROWS_PER_TILE // K):
        pltpu.make_async_copy(rows_v.at[0],
                              acc_sh.at[pl.ds(s * ROWS_PER_TILE, K)],
                              sig).wait()
    plsc.subcore_barrier()

    for ph in range(NPH):
        ib = ph & 1
        if ph + 1 < NPH:
            pltpu.async_copy(gidx.at[wid, ph + 1], gidx_v.at[1 - ib], sig)
            pltpu.async_copy(sidx.at[wid, ph + 1], sidx_v.at[1 - ib], sis)

        def gstart(j, b, ib=ib):
            pltpu.async_copy(table.at[gidx_v.at[ib, j]], rows_v.at[b],
                             gsems[b])

        def gwait(b):
            pltpu.make_async_copy(table.at[gidx_v.at[0, 0]], rows_v.at[b],
                                  gsems[b]).wait()

        def scat(j, b, ib=ib):
            pltpu.sync_copy(rows_v.at[b], acc_sh.at[sidx_v.at[ib, j]],
                            add=True)

        gstart(0, 0)

        def pair(i, carry):
            gstart(2 * i + 1, 1)
            gwait(0)
            scat(2 * i, 0)
            gstart(2 * i + 2, 0)
            gwait(1)
            scat(2 * i + 1, 1)
            return carry

        lax.fori_loop(0, (PB - 1) // 2, pair, 0)
        gwait(0)
        scat(PB - 1, 0)

        if ph + 1 < NPH:
            pltpu.make_async_copy(gidx.at[0, 0], gidx_v.at[1 - ib], sig).wait()
            pltpu.make_async_copy(sidx.at[0, 0], sidx_v.at[1 - ib], sis).wait()

    plsc.subcore_barrier()
    for r in range(ROWS_PER_TILE // K):
        b = r & 1
        sl = pl.ds(s * ROWS_PER_TILE + r * K, K)
        if r >= 2:
            pltpu.make_async_copy(rows_v.at[b], out.at[c, sl],
                                  gsems[b]).wait()
        pltpu.sync_copy(acc_sh.at[sl], rows_v.at[b])
        pltpu.async_copy(rows_v.at[b], out.at[c, sl], gsems[b])
    for b in range(2):
        pltpu.make_async_copy(rows_v.at[b],
                              out.at[c, pl.ds(s * ROWS_PER_TILE, K)],
                              gsems[b]).wait()


def _make_agg():
    mesh = plsc.VectorSubcoreMesh(core_axis_name="c", subcore_axis_name="s")
    return pl.kernel(
        _agg_body,
        out_type=jax.ShapeDtypeStruct((NC, S_PAD, D), jnp.float32),
        mesh=mesh,
        scratch_types=[
            pltpu.VMEM((2, PB, K), jnp.int32),
            pltpu.VMEM((2, PB, K), jnp.int32),
            pltpu.VMEM((2, K, D), jnp.float32),
            pltpu.VMEM_SHARED((S_PAD, D), jnp.float32),
            pltpu.SemaphoreType.DMA,
            pltpu.SemaphoreType.DMA,
            pltpu.SemaphoreType.DMA,
            pltpu.SemaphoreType.DMA,
        ],
    )


def _combine_body(relu, p_ref, cnt_ref, w_ref, b_ref, o_ref):
    ssum = p_ref[0] + p_ref[1]
    cnt = cnt_ref[0] + cnt_ref[1]
    mean = ssum / jnp.maximum(cnt, 1.0)
    y = jnp.dot(mean, w_ref[...], preferred_element_type=jnp.float32)
    y = y[:NE] + b_ref[...][None, :]
    if relu:
        y = jnp.maximum(y, 0.0)
    o_ref[...] = y


def _combine(partials, cnts, w, b, relu):
    body = functools.partial(_combine_body, relu)
    return pl.pallas_call(
        body,
        out_shape=jax.ShapeDtypeStruct((NE, D), jnp.float32),
    )(partials, cnts, w, b)


def _pad_idx(g, s_):
    return (g.reshape(NW, NPH, PB, K), s_.reshape(NW, NPH, PB, K))


def kernel(x, ei, W1_e, b1_e, W1_n, b1_n, W2_e, b2_e, W2_n, b2_n):
    gi_ne, si_ne = _pad_idx(ei[0], ei[1])
    gi_en, si_en = _pad_idx(ei[1], ei[0])
    zeros_b = jnp.zeros((K, D), jnp.float32)
    ones_t = jnp.ones((N, D), jnp.float32)

    agg = _make_agg()
    cnt_e = agg(ones_t, gi_ne, si_ne, zeros_b)
    cnt_n = agg(ones_t, gi_en, si_en, zeros_b)

    h = x
    for (We, be, Wn, bn) in ((W1_e, b1_e, W1_n, b1_n), (W2_e, b2_e, W2_n, b2_n)):
        ep = agg(h, gi_ne, si_ne, zeros_b)
        ef = _combine(ep, cnt_e, We, be, relu=False)
        np_ = agg(ef, gi_en, si_en, zeros_b)
        h = _combine(np_, cnt_n, Wn, bn, relu=True)
    return h
